# Initial kernel scaffold; baseline (speedup 1.0000x reference)
import jax
import jax.numpy as jnp
from jax.experimental import pallas as pl


def kernel(x_initial, edge_index_0, edge_index_1, basic_W1, basic_b1, basic_W2, basic_b2, gnn0_W0, gnn0_b0, gnn0_W1, gnn0_b1, gnn1_W0, gnn1_b0, gnn1_W1, gnn1_b1, dec0_W1, dec0_b1, dec0_W2, dec0_b2, dec1_W1, dec1_b1, dec1_W2, dec1_b2):
    raise NotImplementedError("write your pallas kernel here")


# speedup vs baseline: 7.8199x; 7.8199x over previous
import functools

import jax
import jax.numpy as jnp
from jax import lax
from jax.experimental import pallas as pl
from jax.experimental.pallas import tpu as pltpu
from jax.experimental.pallas import tpu_sc as plsc

N = 10000
D = 256
H = 128
E = 160000
CHUNK = 128
NTILES = 16
NROW = N + 240
RPT = NROW // NTILES
E_PAD = 163840
CPT = E_PAD // NTILES // CHUNK
RBLK = 1000
GRID = N // RBLK

_mesh = plsc.VectorSubcoreMesh(core_axis_name="c", subcore_axis_name="s")


@functools.partial(
    pl.kernel,
    out_type=jax.ShapeDtypeStruct((2, 2, NROW, H), jnp.float32),
    mesh=_mesh,
    scratch_types=[
        pltpu.VMEM((CHUNK,), jnp.int32),
        pltpu.VMEM((CHUNK, H), jnp.float32),
        pltpu.VMEM_SHARED((NROW, H), jnp.float32),
    ],
)
def _deg_kernel(dst0_hbm, dst1_hbm, ones_hbm, zeros_hbm, out_hbm,
                idx_v, ones_v, acc_sh):
    core = lax.axis_index("c")
    sub = lax.axis_index("s")
    r0 = sub * RPT
    cpt = CPT // 2
    pltpu.sync_copy(ones_hbm, ones_v)
    for v, dref in ((0, dst0_hbm), (1, dst1_hbm)):
        pltpu.sync_copy(zeros_hbm.at[pl.ds(r0, RPT)], acc_sh.at[pl.ds(r0, RPT)])
        plsc.subcore_barrier()
        base0 = (core * (E_PAD // 2)) + sub * (cpt * CHUNK)

        @pl.loop(0, cpt)
        def _(ci):
            b = base0 + ci * CHUNK
            pltpu.sync_copy(dref.at[pl.ds(b, CHUNK)], idx_v)
            pltpu.sync_copy(ones_v, acc_sh.at[idx_v], add=True)
        plsc.subcore_barrier()
        pltpu.sync_copy(acc_sh.at[pl.ds(r0, RPT)],
                        out_hbm.at[v, core, pl.ds(r0, RPT)])


@functools.partial(
    pl.kernel,
    out_type=jax.ShapeDtypeStruct((2, 2, NROW, H), jnp.float32),
    mesh=_mesh,
    scratch_types=[
        pltpu.VMEM((CHUNK,), jnp.int32),
        pltpu.VMEM((CHUNK,), jnp.int32),
        pltpu.VMEM((CHUNK, H), jnp.float32),
        pltpu.VMEM_SHARED((NROW, H), jnp.float32),
        pltpu.SemaphoreType.DMA,
    ],
)
def _adj_kernel(srcs0_hbm, dst0_hbm, srcs1_hbm, dst1_hbm, y0_hbm, y1_hbm,
                zeros_hbm, out_hbm, sidx, didx, rows, acc, sem):
    core = lax.axis_index("c")
    sub = lax.axis_index("s")
    r0 = sub * RPT
    for v, (sref, dref, yref) in enumerate(
            ((srcs0_hbm, dst0_hbm, y0_hbm), (srcs1_hbm, dst1_hbm, y1_hbm))):
        pltpu.sync_copy(zeros_hbm.at[pl.ds(r0, RPT)], acc.at[pl.ds(r0, RPT)])
        plsc.subcore_barrier()
        base0 = sub * (CPT * CHUNK)

        @pl.loop(0, CPT)
        def _(ci):
            b = base0 + ci * CHUNK
            pltpu.sync_copy(sref.at[core, pl.ds(b, CHUNK)], sidx)
            pltpu.sync_copy(dref.at[pl.ds(b, CHUNK)], didx)
            pltpu.async_copy(yref.at[sidx], rows, sem).wait()
            pltpu.sync_copy(rows, acc.at[didx], add=True)
        plsc.subcore_barrier()
        pltpu.sync_copy(acc.at[pl.ds(r0, RPT)],
                        out_hbm.at[v, core, pl.ds(r0, RPT)])


def _row_spec(cols):
    return pl.BlockSpec((RBLK, cols), lambda i: (i, 0))


def _stk_spec():
    return pl.BlockSpec((2, RBLK, H), lambda i: (0, i, 0))


def _w_spec():
    return pl.BlockSpec((D, D), lambda i: (0, 0))


def _b_spec():
    return pl.BlockSpec((1, D), lambda i: (0, 0))


def _tc1_body(x_ref, deg0_ref, deg1_ref, bW1_ref, bb1_ref, bW2_ref, bb2_ref,
              g0W_ref, g1W_ref, xb_ref, y0_ref, y1_ref, dinv0_ref, dinv1_ref):
    x = x_ref[...]
    dinv0 = lax.rsqrt(deg0_ref[...] + 1.0)
    dinv1 = lax.rsqrt(deg1_ref[...] + 1.0)
    dinv0_ref[...] = dinv0
    dinv1_ref[...] = dinv1
    h = jnp.maximum(jnp.dot(x, bW1_ref[...]) + bb1_ref[...], 0.0)
    xb_ref[...] = jnp.maximum(jnp.dot(h, bW2_ref[...]) + bb2_ref[...], 0.0)
    y0 = dinv0[:, :1] * jnp.dot(x, g0W_ref[...])
    y1 = dinv1[:, :1] * jnp.dot(x, g1W_ref[...])
    y0_ref[0] = y0[:, :H]
    y0_ref[1] = y0[:, H:]
    y1_ref[0] = y1[:, :H]
    y1_ref[1] = y1[:, H:]


def _tc2_body(A0a_ref, A0b_ref, A1a_ref, A1b_ref, y0_ref, y1_ref,
              dinv0_ref, dinv1_ref, g0b0_ref, g1b0_ref, g0W1_ref, g1W1_ref,
              y0n_ref, y1n_ref):
    for (Aa, Ab, y, dv, b, W, yn) in (
            (A0a_ref, A0b_ref, y0_ref, dinv0_ref, g0b0_ref, g0W1_ref, y0n_ref),
            (A1a_ref, A1b_ref, y1_ref, dinv1_ref, g1b0_ref, g1W1_ref, y1n_ref)):
        d = dv[...][:, :1]
        A = jnp.concatenate([Aa[...] + y[0], Ab[...] + y[1]], axis=1)
        hcur = jnp.maximum(d * A + b[...], 0.0)
        ynew = d * jnp.dot(hcur, W[...])
        yn[0] = ynew[:, :H]
        yn[1] = ynew[:, H:]


def _tc3_body(A0a_ref, A0b_ref, A1a_ref, A1b_ref, y0_ref, y1_ref,
              dinv0_ref, dinv1_ref, g0b1_ref, g1b1_ref,
              d0W1_ref, d0b1_ref, d0W2_ref, d0b2_ref,
              d1W1_ref, d1b1_ref, d1W2_ref, d1b2_ref, xb_ref, out_ref):
    acc = xb_ref[...]
    for (Aa, Ab, y, dv, gb, W1, b1, W2, b2) in (
            (A0a_ref, A0b_ref, y0_ref, dinv0_ref, g0b1_ref,
             d0W1_ref, d0b1_ref, d0W2_ref, d0b2_ref),
            (A1a_ref, A1b_ref, y1_ref, dinv1_ref, g1b1_ref,
             d1W1_ref, d1b1_ref, d1W2_ref, d1b2_ref)):
        d = dv[...][:, :1]
        A = jnp.concatenate([Aa[...] + y[0], Ab[...] + y[1]], axis=1)
        Hcur = jnp.maximum(d * A + gb[...], 0.0)
        h1 = jnp.maximum(jnp.dot(Hcur, W1[...]) + b1[...], 0.0)
        acc = acc + jnp.maximum(jnp.dot(h1, W2[...]) + b2[...], 0.0)
    out_ref[...] = acc


def kernel(x_initial, edge_index_0, edge_index_1,
           basic_W1, basic_b1, basic_W2, basic_b2,
           gnn0_W0, gnn0_b0, gnn0_W1, gnn0_b1,
           gnn1_W0, gnn1_b0, gnn1_W1, gnn1_b1,
           dec0_W1, dec0_b1, dec0_W2, dec0_b2,
           dec1_W1, dec1_b1, dec1_W2, dec1_b2):
    f32 = jnp.float32
    pad = E_PAD - E
    pad_src = (jnp.arange(pad, dtype=jnp.int32) * 997) % N
    pad_dst = N + (jnp.arange(pad, dtype=jnp.int32) % 240)

    def prep(ei):
        s = jnp.concatenate([ei[0].astype(jnp.int32), pad_src])
        dd = jnp.concatenate([ei[1].astype(jnp.int32), pad_dst])
        return jnp.stack([s, s + N]), dd

    srcs0, dst0 = prep(edge_index_0)
    srcs1, dst1 = prep(edge_index_1)

    onesH = jnp.ones((CHUNK, H), f32)
    zerosH = jnp.zeros((NROW, H), f32)

    deg_out = _deg_kernel(dst0, dst1, onesH, zerosH)
    deg0 = deg_out[0, 0, :N, :16] + deg_out[0, 1, :N, :16]
    deg1 = deg_out[1, 0, :N, :16] + deg_out[1, 1, :N, :16]

    b = lambda v: v.reshape(1, D)
    row_f = jax.ShapeDtypeStruct((N, D), f32)
    stk_f = jax.ShapeDtypeStruct((2, N, H), f32)
    d16_f = jax.ShapeDtypeStruct((N, 16), f32)

    xb, y0, y1, dinv0, dinv1 = pl.pallas_call(
        _tc1_body,
        grid=(GRID,),
        in_specs=[_row_spec(D), _row_spec(16), _row_spec(16),
                  _w_spec(), _b_spec(), _w_spec(), _b_spec(),
                  _w_spec(), _w_spec()],
        out_specs=[_row_spec(D), _stk_spec(), _stk_spec(),
                   _row_spec(16), _row_spec(16)],
        out_shape=[row_f, stk_f, stk_f, d16_f, d16_f],
    )(x_initial, deg0, deg1, basic_W1, b(basic_b1), basic_W2, b(basic_b2),
      gnn0_W0, gnn1_W0)

    def adj(y0s, y1s):
        out = _adj_kernel(srcs0, dst0, srcs1, dst1,
                          y0s.reshape(2 * N, H), y1s.reshape(2 * N, H), zerosH)
        return (out[0, 0, :N, :], out[0, 1, :N, :],
                out[1, 0, :N, :], out[1, 1, :N, :])

    A0a, A0b, A1a, A1b = adj(y0, y1)

    y0n, y1n = pl.pallas_call(
        _tc2_body,
        grid=(GRID,),
        in_specs=[_row_spec(H)] * 4 + [_stk_spec(), _stk_spec(),
                  _row_spec(16), _row_spec(16),
                  _b_spec(), _b_spec(), _w_spec(), _w_spec()],
        out_specs=[_stk_spec(), _stk_spec()],
        out_shape=[stk_f, stk_f],
    )(A0a, A0b, A1a, A1b, y0, y1, dinv0, dinv1,
      b(gnn0_b0), b(gnn1_b0), gnn0_W1, gnn1_W1)

    B0a, B0b, B1a, B1b = adj(y0n, y1n)

    x_total = pl.pallas_call(
        _tc3_body,
        grid=(GRID,),
        in_specs=[_row_spec(H)] * 4 + [_stk_spec(), _stk_spec(),
                  _row_spec(16), _row_spec(16), _b_spec(), _b_spec(),
                  _w_spec(), _b_spec(), _w_spec(), _b_spec(),
                  _w_spec(), _b_spec(), _w_spec(), _b_spec(),
                  _row_spec(D)],
        out_specs=_row_spec(D),
        out_shape=row_f,
    )(B0a, B0b, B1a, B1b, y0n, y1n, dinv0, dinv1,
      b(gnn0_b1), b(gnn1_b1),
      dec0_W1, b(dec0_b1), dec0_W2, b(dec0_b2),
      dec1_W1, b(dec1_b1), dec1_W2, b(dec1_b2), xb)

    return (x_total, jnp.float32(0.0))


# speedup vs baseline: 10.2138x; 1.3061x over previous
import functools

import jax
import jax.numpy as jnp
from jax import lax
from jax.experimental import pallas as pl
from jax.experimental.pallas import tpu as pltpu
from jax.experimental.pallas import tpu_sc as plsc

N = 10000
D = 256
H = 128
E = 160000
CHUNK = 128
NTILES = 16
NROW = N + 240
RPT = NROW // NTILES
E_PAD = 163840
NCH = E_PAD // CHUNK
CPT = NCH // NTILES
G = CPT // 2
RBLK = 1000
GRID = N // RBLK

_mesh = plsc.VectorSubcoreMesh(core_axis_name="c", subcore_axis_name="s")


@functools.partial(
    pl.kernel,
    out_type=jax.ShapeDtypeStruct((2, 2, NROW, H), jnp.float32),
    mesh=_mesh,
    scratch_types=[
        pltpu.VMEM((CHUNK,), jnp.int32),
        pltpu.VMEM((CHUNK, H), jnp.float32),
        pltpu.VMEM_SHARED((NROW, H), jnp.float32),
    ],
)
def _deg_kernel(dst0_hbm, dst1_hbm, ones_hbm, zeros_hbm, out_hbm,
                idx_v, ones_v, acc_sh):
    core = lax.axis_index("c")
    sub = lax.axis_index("s")
    r0 = sub * RPT
    cpt = CPT // 2
    pltpu.sync_copy(ones_hbm, ones_v)
    for v, dref in ((0, dst0_hbm), (1, dst1_hbm)):
        pltpu.sync_copy(zeros_hbm.at[pl.ds(r0, RPT)], acc_sh.at[pl.ds(r0, RPT)])
        plsc.subcore_barrier()
        base0 = (core * (E_PAD // 2)) + sub * (cpt * CHUNK)

        @pl.loop(0, cpt)
        def _(ci):
            b = base0 + ci * CHUNK
            pltpu.sync_copy(dref.at[pl.ds(b, CHUNK)], idx_v)
            pltpu.sync_copy(ones_v, acc_sh.at[idx_v], add=True)
        plsc.subcore_barrier()
        pltpu.sync_copy(acc_sh.at[pl.ds(r0, RPT)],
                        out_hbm.at[v, core, pl.ds(r0, RPT)])


@functools.partial(
    pl.kernel,
    out_type=jax.ShapeDtypeStruct((2, 2, NROW, H), jnp.float32),
    mesh=_mesh,
    scratch_types=[
        pltpu.VMEM((CHUNK,), jnp.int32),
        pltpu.VMEM((CHUNK,), jnp.int32),
        pltpu.VMEM((CHUNK,), jnp.int32),
        pltpu.VMEM((CHUNK,), jnp.int32),
        pltpu.VMEM((CHUNK, H), jnp.float32),
        pltpu.VMEM((CHUNK, H), jnp.float32),
        pltpu.VMEM_SHARED((NROW, H), jnp.float32),
        pltpu.SemaphoreType.DMA,
        pltpu.SemaphoreType.DMA,
    ],
)
def _adj_kernel(srcs0_hbm, dst0_hbm, srcs1_hbm, dst1_hbm, y0_hbm, y1_hbm,
                zeros_hbm, out_hbm, sidx0, didx0, sidx1, didx1,
                rows0, rows1, acc, sem0, sem1):
    core = lax.axis_index("c")
    sub = lax.axis_index("s")
    r0 = sub * RPT
    bufs = ((sidx0, didx0, rows0, sem0), (sidx1, didx1, rows1, sem1))
    for v, (sref, dref, yref) in enumerate(
            ((srcs0_hbm, dst0_hbm, y0_hbm), (srcs1_hbm, dst1_hbm, y1_hbm))):
        pltpu.sync_copy(zeros_hbm.at[pl.ds(r0, RPT)], acc.at[pl.ds(r0, RPT)])
        plsc.subcore_barrier()
        base0 = sub * (CPT * CHUNK)

        def issue(b, bi):
            si, di, ro, se = bufs[bi]
            pltpu.sync_copy(sref.at[core, pl.ds(b, CHUNK)], si)
            pltpu.sync_copy(dref.at[pl.ds(b, CHUNK)], di)
            return pltpu.async_copy(yref.at[si], ro, se)

        @pl.loop(0, CPT, step=2)
        def _(ci):
            b = base0 + ci * CHUNK
            h0 = issue(b, 0)
            h1 = issue(b + CHUNK, 1)
            h0.wait()
            pltpu.sync_copy(rows0, acc.at[didx0], add=True)
            h1.wait()
            pltpu.sync_copy(rows1, acc.at[didx1], add=True)
        plsc.subcore_barrier()
        pltpu.sync_copy(acc.at[pl.ds(r0, RPT)],
                        out_hbm.at[v, core, pl.ds(r0, RPT)])


def _row_spec(cols):
    return pl.BlockSpec((RBLK, cols), lambda i: (i, 0))


def _stk_spec():
    return pl.BlockSpec((2, RBLK, H), lambda i: (0, i, 0))


def _w_spec():
    return pl.BlockSpec((D, D), lambda i: (0, 0))


def _b_spec():
    return pl.BlockSpec((1, D), lambda i: (0, 0))


def _tc1_body(x_ref, deg0_ref, deg1_ref, bW1_ref, bb1_ref, bW2_ref, bb2_ref,
              g0W_ref, g1W_ref, xb_ref, y0_ref, y1_ref, dinv0_ref, dinv1_ref):
    x = x_ref[...]
    dinv0 = lax.rsqrt(deg0_ref[...] + 1.0)
    dinv1 = lax.rsqrt(deg1_ref[...] + 1.0)
    dinv0_ref[...] = dinv0
    dinv1_ref[...] = dinv1
    h = jnp.maximum(jnp.dot(x, bW1_ref[...]) + bb1_ref[...], 0.0)
    xb_ref[...] = jnp.maximum(jnp.dot(h, bW2_ref[...]) + bb2_ref[...], 0.0)
    y0 = dinv0[:, :1] * jnp.dot(x, g0W_ref[...])
    y1 = dinv1[:, :1] * jnp.dot(x, g1W_ref[...])
    y0_ref[0] = y0[:, :H]
    y0_ref[1] = y0[:, H:]
    y1_ref[0] = y1[:, :H]
    y1_ref[1] = y1[:, H:]


def _tc2_body(A0a_ref, A0b_ref, A1a_ref, A1b_ref, y0_ref, y1_ref,
              dinv0_ref, dinv1_ref, g0b0_ref, g1b0_ref, g0W1_ref, g1W1_ref,
              y0n_ref, y1n_ref):
    for (Aa, Ab, y, dv, b, W, yn) in (
            (A0a_ref, A0b_ref, y0_ref, dinv0_ref, g0b0_ref, g0W1_ref, y0n_ref),
            (A1a_ref, A1b_ref, y1_ref, dinv1_ref, g1b0_ref, g1W1_ref, y1n_ref)):
        d = dv[...][:, :1]
        A = jnp.concatenate([Aa[...] + y[0], Ab[...] + y[1]], axis=1)
        hcur = jnp.maximum(d * A + b[...], 0.0)
        ynew = d * jnp.dot(hcur, W[...])
        yn[0] = ynew[:, :H]
        yn[1] = ynew[:, H:]


def _tc3_body(A0a_ref, A0b_ref, A1a_ref, A1b_ref, y0_ref, y1_ref,
              dinv0_ref, dinv1_ref, g0b1_ref, g1b1_ref,
              d0W1_ref, d0b1_ref, d0W2_ref, d0b2_ref,
              d1W1_ref, d1b1_ref, d1W2_ref, d1b2_ref, xb_ref, out_ref):
    acc = xb_ref[...]
    for (Aa, Ab, y, dv, gb, W1, b1, W2, b2) in (
            (A0a_ref, A0b_ref, y0_ref, dinv0_ref, g0b1_ref,
             d0W1_ref, d0b1_ref, d0W2_ref, d0b2_ref),
            (A1a_ref, A1b_ref, y1_ref, dinv1_ref, g1b1_ref,
             d1W1_ref, d1b1_ref, d1W2_ref, d1b2_ref)):
        d = dv[...][:, :1]
        A = jnp.concatenate([Aa[...] + y[0], Ab[...] + y[1]], axis=1)
        Hcur = jnp.maximum(d * A + gb[...], 0.0)
        h1 = jnp.maximum(jnp.dot(Hcur, W1[...]) + b1[...], 0.0)
        acc = acc + jnp.maximum(jnp.dot(h1, W2[...]) + b2[...], 0.0)
    out_ref[...] = acc


def kernel(x_initial, edge_index_0, edge_index_1,
           basic_W1, basic_b1, basic_W2, basic_b2,
           gnn0_W0, gnn0_b0, gnn0_W1, gnn0_b1,
           gnn1_W0, gnn1_b0, gnn1_W1, gnn1_b1,
           dec0_W1, dec0_b1, dec0_W2, dec0_b2,
           dec1_W1, dec1_b1, dec1_W2, dec1_b2):
    f32 = jnp.float32
    pad = E_PAD - E
    pad_src = (jnp.arange(pad, dtype=jnp.int32) * 997) % N
    pad_dst = N + (jnp.arange(pad, dtype=jnp.int32) % 240)

    def prep(ei):
        s = jnp.concatenate([ei[0].astype(jnp.int32), pad_src])
        dd = jnp.concatenate([ei[1].astype(jnp.int32), pad_dst])
        return jnp.stack([s, s + N]), dd

    srcs0, dst0 = prep(edge_index_0)
    srcs1, dst1 = prep(edge_index_1)

    onesH = jnp.ones((CHUNK, H), f32)
    zerosH = jnp.zeros((NROW, H), f32)

    deg_out = _deg_kernel(dst0, dst1, onesH, zerosH)
    deg0 = deg_out[0, 0, :N, :16] + deg_out[0, 1, :N, :16]
    deg1 = deg_out[1, 0, :N, :16] + deg_out[1, 1, :N, :16]

    b = lambda v: v.reshape(1, D)
    row_f = jax.ShapeDtypeStruct((N, D), f32)
    stk_f = jax.ShapeDtypeStruct((2, N, H), f32)
    d16_f = jax.ShapeDtypeStruct((N, 16), f32)

    xb, y0, y1, dinv0, dinv1 = pl.pallas_call(
        _tc1_body,
        grid=(GRID,),
        in_specs=[_row_spec(D), _row_spec(16), _row_spec(16),
                  _w_spec(), _b_spec(), _w_spec(), _b_spec(),
                  _w_spec(), _w_spec()],
        out_specs=[_row_spec(D), _stk_spec(), _stk_spec(),
                   _row_spec(16), _row_spec(16)],
        out_shape=[row_f, stk_f, stk_f, d16_f, d16_f],
    )(x_initial, deg0, deg1, basic_W1, b(basic_b1), basic_W2, b(basic_b2),
      gnn0_W0, gnn1_W0)

    def adj(y0s, y1s):
        out = _adj_kernel(srcs0, dst0, srcs1, dst1,
                          y0s.reshape(2 * N, H), y1s.reshape(2 * N, H), zerosH)
        return (out[0, 0, :N, :], out[0, 1, :N, :],
                out[1, 0, :N, :], out[1, 1, :N, :])

    A0a, A0b, A1a, A1b = adj(y0, y1)

    y0n, y1n = pl.pallas_call(
        _tc2_body,
        grid=(GRID,),
        in_specs=[_row_spec(H)] * 4 + [_stk_spec(), _stk_spec(),
                  _row_spec(16), _row_spec(16),
                  _b_spec(), _b_spec(), _w_spec(), _w_spec()],
        out_specs=[_stk_spec(), _stk_spec()],
        out_shape=[stk_f, stk_f],
    )(A0a, A0b, A1a, A1b, y0, y1, dinv0, dinv1,
      b(gnn0_b0), b(gnn1_b0), gnn0_W1, gnn1_W1)

    B0a, B0b, B1a, B1b = adj(y0n, y1n)

    x_total = pl.pallas_call(
        _tc3_body,
        grid=(GRID,),
        in_specs=[_row_spec(H)] * 4 + [_stk_spec(), _stk_spec(),
                  _row_spec(16), _row_spec(16), _b_spec(), _b_spec(),
                  _w_spec(), _b_spec(), _w_spec(), _b_spec(),
                  _w_spec(), _b_spec(), _w_spec(), _b_spec(),
                  _row_spec(D)],
        out_specs=_row_spec(D),
        out_shape=row_f,
    )(B0a, B0b, B1a, B1b, y0n, y1n, dinv0, dinv1,
      b(gnn0_b1), b(gnn1_b1),
      dec0_W1, b(dec0_b1), dec0_W2, b(dec0_b2),
      dec1_W1, b(dec1_b1), dec1_W2, b(dec1_b2), xb)

    return (x_total, jnp.float32(0.0))


# speedup vs baseline: 11.4646x; 1.1225x over previous
import functools

import jax
import jax.numpy as jnp
from jax import lax
from jax.experimental import pallas as pl
from jax.experimental.pallas import tpu as pltpu
from jax.experimental.pallas import tpu_sc as plsc

N = 10000
D = 256
H = 128
E = 160000
CHUNK = 128
NTILES = 16
NROW = N + 240
RPT = NROW // NTILES
E_PAD = 163840
NCH = E_PAD // CHUNK
CPT = NCH // NTILES
G = CPT // 2
RBLK = 1000
GRID = N // RBLK
DCH = 1024
DPT = (E_PAD // 2) // NTILES // DCH

_mesh = plsc.VectorSubcoreMesh(core_axis_name="c", subcore_axis_name="s")


@functools.partial(
    pl.kernel,
    out_type=jax.ShapeDtypeStruct((2, 2, NROW), jnp.float32),
    mesh=_mesh,
    scratch_types=[
        pltpu.VMEM((DCH,), jnp.int32),
        pltpu.VMEM((DCH,), jnp.float32),
        pltpu.VMEM_SHARED((NROW,), jnp.float32),
    ],
)
def _deg_kernel(dst0_hbm, dst1_hbm, ones_hbm, zeros_hbm, out_hbm,
                idx_v, ones_v, acc_sh):
    core = lax.axis_index("c")
    sub = lax.axis_index("s")
    r0 = sub * RPT
    pltpu.sync_copy(ones_hbm, ones_v)
    for v, dref in ((0, dst0_hbm), (1, dst1_hbm)):
        pltpu.sync_copy(zeros_hbm.at[pl.ds(r0, RPT)], acc_sh.at[pl.ds(r0, RPT)])
        plsc.subcore_barrier()
        base0 = (core * (E_PAD // 2)) + sub * (DPT * DCH)

        @pl.loop(0, DPT)
        def _(ci):
            b = base0 + ci * DCH
            pltpu.sync_copy(dref.at[pl.ds(b, DCH)], idx_v)
            pltpu.sync_copy(ones_v, acc_sh.at[idx_v], add=True)
        plsc.subcore_barrier()
        pltpu.sync_copy(acc_sh.at[pl.ds(r0, RPT)],
                        out_hbm.at[v, core, pl.ds(r0, RPT)])


@functools.partial(
    pl.kernel,
    out_type=jax.ShapeDtypeStruct((2, 2, NROW, H), jnp.float32),
    mesh=_mesh,
    scratch_types=[
        pltpu.VMEM((CHUNK,), jnp.int32),
        pltpu.VMEM((CHUNK,), jnp.int32),
        pltpu.VMEM((CHUNK,), jnp.int32),
        pltpu.VMEM((CHUNK,), jnp.int32),
        pltpu.VMEM((CHUNK, H), jnp.float32),
        pltpu.VMEM((CHUNK, H), jnp.float32),
        pltpu.VMEM_SHARED((NROW, H), jnp.float32),
        pltpu.SemaphoreType.DMA,
        pltpu.SemaphoreType.DMA,
    ],
)
def _adj_kernel(srcs0_hbm, dst0_hbm, srcs1_hbm, dst1_hbm, y0_hbm, y1_hbm,
                zeros_hbm, out_hbm, sidx0, didx0, sidx1, didx1,
                rows0, rows1, acc, sem0, sem1):
    core = lax.axis_index("c")
    sub = lax.axis_index("s")
    r0 = sub * RPT
    bufs = ((sidx0, didx0, rows0, sem0), (sidx1, didx1, rows1, sem1))
    for v, (sref, dref, yref) in enumerate(
            ((srcs0_hbm, dst0_hbm, y0_hbm), (srcs1_hbm, dst1_hbm, y1_hbm))):
        pltpu.sync_copy(zeros_hbm.at[pl.ds(r0, RPT)], acc.at[pl.ds(r0, RPT)])
        plsc.subcore_barrier()
        base0 = sub * (CPT * CHUNK)

        def issue(b, bi):
            si, di, ro, se = bufs[bi]
            pltpu.sync_copy(sref.at[core, pl.ds(b, CHUNK)], si)
            pltpu.sync_copy(dref.at[pl.ds(b, CHUNK)], di)
            return pltpu.async_copy(yref.at[si], ro, se)

        @pl.loop(0, CPT, step=2)
        def _(ci):
            b = base0 + ci * CHUNK
            h0 = issue(b, 0)
            h1 = issue(b + CHUNK, 1)
            h0.wait()
            pltpu.sync_copy(rows0, acc.at[didx0], add=True)
            h1.wait()
            pltpu.sync_copy(rows1, acc.at[didx1], add=True)
        plsc.subcore_barrier()
        pltpu.sync_copy(acc.at[pl.ds(r0, RPT)],
                        out_hbm.at[v, core, pl.ds(r0, RPT)])


def _row_spec(cols):
    return pl.BlockSpec((RBLK, cols), lambda i: (i, 0))


def _stk_spec():
    return pl.BlockSpec((2, RBLK, H), lambda i: (0, i, 0))


def _w_spec():
    return pl.BlockSpec((D, D), lambda i: (0, 0))


def _b_spec():
    return pl.BlockSpec((1, D), lambda i: (0, 0))


def _tc1_body(x_ref, deg0_ref, deg1_ref, bW1_ref, bb1_ref, bW2_ref, bb2_ref,
              g0W_ref, g1W_ref, xb_ref, y0_ref, y1_ref, dinv0_ref, dinv1_ref):
    x = x_ref[...]
    dinv0 = lax.rsqrt(deg0_ref[...] + 1.0)
    dinv1 = lax.rsqrt(deg1_ref[...] + 1.0)
    dinv0_ref[...] = dinv0
    dinv1_ref[...] = dinv1
    h = jnp.maximum(jnp.dot(x, bW1_ref[...]) + bb1_ref[...], 0.0)
    xb_ref[...] = jnp.maximum(jnp.dot(h, bW2_ref[...]) + bb2_ref[...], 0.0)
    y0 = dinv0 * jnp.dot(x, g0W_ref[...])
    y1 = dinv1 * jnp.dot(x, g1W_ref[...])
    y0_ref[0] = y0[:, :H]
    y0_ref[1] = y0[:, H:]
    y1_ref[0] = y1[:, :H]
    y1_ref[1] = y1[:, H:]


def _tc2_body(A0a_ref, A0b_ref, A1a_ref, A1b_ref, y0_ref, y1_ref,
              dinv0_ref, dinv1_ref, g0b0_ref, g1b0_ref, g0W1_ref, g1W1_ref,
              y0n_ref, y1n_ref):
    for (Aa, Ab, y, dv, b, W, yn) in (
            (A0a_ref, A0b_ref, y0_ref, dinv0_ref, g0b0_ref, g0W1_ref, y0n_ref),
            (A1a_ref, A1b_ref, y1_ref, dinv1_ref, g1b0_ref, g1W1_ref, y1n_ref)):
        d = dv[...]
        A = jnp.concatenate([Aa[...] + y[0], Ab[...] + y[1]], axis=1)
        hcur = jnp.maximum(d * A + b[...], 0.0)
        ynew = d * jnp.dot(hcur, W[...])
        yn[0] = ynew[:, :H]
        yn[1] = ynew[:, H:]


def _tc3_body(A0a_ref, A0b_ref, A1a_ref, A1b_ref, y0_ref, y1_ref,
              dinv0_ref, dinv1_ref, g0b1_ref, g1b1_ref,
              d0W1_ref, d0b1_ref, d0W2_ref, d0b2_ref,
              d1W1_ref, d1b1_ref, d1W2_ref, d1b2_ref, xb_ref, out_ref):
    acc = xb_ref[...]
    for (Aa, Ab, y, dv, gb, W1, b1, W2, b2) in (
            (A0a_ref, A0b_ref, y0_ref, dinv0_ref, g0b1_ref,
             d0W1_ref, d0b1_ref, d0W2_ref, d0b2_ref),
            (A1a_ref, A1b_ref, y1_ref, dinv1_ref, g1b1_ref,
             d1W1_ref, d1b1_ref, d1W2_ref, d1b2_ref)):
        d = dv[...]
        A = jnp.concatenate([Aa[...] + y[0], Ab[...] + y[1]], axis=1)
        Hcur = jnp.maximum(d * A + gb[...], 0.0)
        h1 = jnp.maximum(jnp.dot(Hcur, W1[...]) + b1[...], 0.0)
        acc = acc + jnp.maximum(jnp.dot(h1, W2[...]) + b2[...], 0.0)
    out_ref[...] = acc


def kernel(x_initial, edge_index_0, edge_index_1,
           basic_W1, basic_b1, basic_W2, basic_b2,
           gnn0_W0, gnn0_b0, gnn0_W1, gnn0_b1,
           gnn1_W0, gnn1_b0, gnn1_W1, gnn1_b1,
           dec0_W1, dec0_b1, dec0_W2, dec0_b2,
           dec1_W1, dec1_b1, dec1_W2, dec1_b2):
    f32 = jnp.float32
    pad = E_PAD - E
    pad_src = (jnp.arange(pad, dtype=jnp.int32) * 997) % N
    pad_dst = N + (jnp.arange(pad, dtype=jnp.int32) % 240)

    def prep(ei):
        s = jnp.concatenate([ei[0].astype(jnp.int32), pad_src])
        dd = jnp.concatenate([ei[1].astype(jnp.int32), pad_dst])
        return jnp.stack([s, s + N]), dd

    srcs0, dst0 = prep(edge_index_0)
    srcs1, dst1 = prep(edge_index_1)

    zerosH = jnp.zeros((NROW, H), f32)
    ones1 = jnp.ones((DCH,), f32)
    zeros1 = jnp.zeros((NROW,), f32)

    deg_out = _deg_kernel(dst0, dst1, ones1, zeros1)
    deg0 = (deg_out[0, 0, :N] + deg_out[0, 1, :N])[:, None]
    deg1 = (deg_out[1, 0, :N] + deg_out[1, 1, :N])[:, None]

    b = lambda v: v.reshape(1, D)
    row_f = jax.ShapeDtypeStruct((N, D), f32)
    stk_f = jax.ShapeDtypeStruct((2, N, H), f32)
    d1_f = jax.ShapeDtypeStruct((N, 1), f32)

    xb, y0, y1, dinv0, dinv1 = pl.pallas_call(
        _tc1_body,
        grid=(GRID,),
        in_specs=[_row_spec(D), _row_spec(1), _row_spec(1),
                  _w_spec(), _b_spec(), _w_spec(), _b_spec(),
                  _w_spec(), _w_spec()],
        out_specs=[_row_spec(D), _stk_spec(), _stk_spec(),
                   _row_spec(1), _row_spec(1)],
        out_shape=[row_f, stk_f, stk_f, d1_f, d1_f],
    )(x_initial, deg0, deg1, basic_W1, b(basic_b1), basic_W2, b(basic_b2),
      gnn0_W0, gnn1_W0)

    def adj(y0s, y1s):
        out = _adj_kernel(srcs0, dst0, srcs1, dst1,
                          y0s.reshape(2 * N, H), y1s.reshape(2 * N, H), zerosH)
        return (out[0, 0, :N, :], out[0, 1, :N, :],
                out[1, 0, :N, :], out[1, 1, :N, :])

    A0a, A0b, A1a, A1b = adj(y0, y1)

    y0n, y1n = pl.pallas_call(
        _tc2_body,
        grid=(GRID,),
        in_specs=[_row_spec(H)] * 4 + [_stk_spec(), _stk_spec(),
                  _row_spec(1), _row_spec(1),
                  _b_spec(), _b_spec(), _w_spec(), _w_spec()],
        out_specs=[_stk_spec(), _stk_spec()],
        out_shape=[stk_f, stk_f],
    )(A0a, A0b, A1a, A1b, y0, y1, dinv0, dinv1,
      b(gnn0_b0), b(gnn1_b0), gnn0_W1, gnn1_W1)

    B0a, B0b, B1a, B1b = adj(y0n, y1n)

    x_total = pl.pallas_call(
        _tc3_body,
        grid=(GRID,),
        in_specs=[_row_spec(H)] * 4 + [_stk_spec(), _stk_spec(),
                  _row_spec(1), _row_spec(1), _b_spec(), _b_spec(),
                  _w_spec(), _b_spec(), _w_spec(), _b_spec(),
                  _w_spec(), _b_spec(), _w_spec(), _b_spec(),
                  _row_spec(D)],
        out_specs=_row_spec(D),
        out_shape=row_f,
    )(B0a, B0b, B1a, B1b, y0n, y1n, dinv0, dinv1,
      b(gnn0_b1), b(gnn1_b1),
      dec0_W1, b(dec0_b1), dec0_W2, b(dec0_b2),
      dec1_W1, b(dec1_b1), dec1_W2, b(dec1_b2), xb)

    return (x_total, jnp.float32(0.0))


# speedup vs baseline: 13.0196x; 1.1356x over previous
import functools

import jax
import jax.numpy as jnp
from jax import lax
from jax.experimental import pallas as pl
from jax.experimental.pallas import tpu as pltpu
from jax.experimental.pallas import tpu_sc as plsc

N = 10000
D = 256
H = 128
E = 160000
CHUNK = 128
NTILES = 16
NROW = N + 240
RPT = NROW // NTILES
E_PAD = 163840
NCH = E_PAD // CHUNK
CPT = NCH // NTILES
G = CPT // 2
RBLK = 1000
GRID = N // RBLK
DCH = 1024
DPT = (E_PAD // 2) // NTILES // DCH

_mesh = plsc.VectorSubcoreMesh(core_axis_name="c", subcore_axis_name="s")


@functools.partial(
    pl.kernel,
    out_type=jax.ShapeDtypeStruct((2, 2, NROW), jnp.float32),
    mesh=_mesh,
    scratch_types=[
        pltpu.VMEM((DCH,), jnp.int32),
        pltpu.VMEM((DCH,), jnp.float32),
        pltpu.VMEM_SHARED((NROW,), jnp.float32),
    ],
)
def _deg_kernel(dst0_hbm, dst1_hbm, ones_hbm, zeros_hbm, out_hbm,
                idx_v, ones_v, acc_sh):
    core = lax.axis_index("c")
    sub = lax.axis_index("s")
    r0 = sub * RPT
    pltpu.sync_copy(ones_hbm, ones_v)
    for v, dref in ((0, dst0_hbm), (1, dst1_hbm)):
        pltpu.sync_copy(zeros_hbm.at[pl.ds(r0, RPT)], acc_sh.at[pl.ds(r0, RPT)])
        plsc.subcore_barrier()
        base0 = (core * (E_PAD // 2)) + sub * (DPT * DCH)

        @pl.loop(0, DPT)
        def _(ci):
            b = base0 + ci * DCH
            pltpu.sync_copy(dref.at[pl.ds(b, DCH)], idx_v)
            pltpu.sync_copy(ones_v, acc_sh.at[idx_v], add=True)
        plsc.subcore_barrier()
        pltpu.sync_copy(acc_sh.at[pl.ds(r0, RPT)],
                        out_hbm.at[v, core, pl.ds(r0, RPT)])


@functools.partial(
    pl.kernel,
    out_type=jax.ShapeDtypeStruct((2, 2, NROW, H), jnp.float32),
    mesh=_mesh,
    scratch_types=[
        pltpu.VMEM((CHUNK,), jnp.int32),
        pltpu.VMEM((CHUNK,), jnp.int32),
        pltpu.VMEM((CHUNK,), jnp.int32),
        pltpu.VMEM((CHUNK,), jnp.int32),
        pltpu.VMEM((CHUNK, H), jnp.float32),
        pltpu.VMEM((CHUNK, H), jnp.float32),
        pltpu.VMEM_SHARED((NROW, H), jnp.float32),
        pltpu.SemaphoreType.DMA,
        pltpu.SemaphoreType.DMA,
    ],
)
def _adj_kernel(srcs0_hbm, dst0_hbm, srcs1_hbm, dst1_hbm, y0_hbm, y1_hbm,
                zeros_hbm, out_hbm, sidx0, didx0, sidx1, didx1,
                rows0, rows1, acc, sem0, sem1):
    core = lax.axis_index("c")
    sub = lax.axis_index("s")
    r0 = sub * RPT
    bufs = ((sidx0, didx0, rows0, sem0), (sidx1, didx1, rows1, sem1))
    for v, (sref, dref, yref) in enumerate(
            ((srcs0_hbm, dst0_hbm, y0_hbm), (srcs1_hbm, dst1_hbm, y1_hbm))):
        pltpu.sync_copy(zeros_hbm.at[pl.ds(r0, RPT)], acc.at[pl.ds(r0, RPT)])
        plsc.subcore_barrier()
        base0 = sub * (CPT * CHUNK)

        def issue(b, bi):
            si, di, ro, se = bufs[bi]
            pltpu.sync_copy(sref.at[core, pl.ds(b, CHUNK)], si)
            pltpu.sync_copy(dref.at[pl.ds(b, CHUNK)], di)
            pltpu.async_copy(yref.at[si], ro, se)

        def drain_scatter(bi):
            si, di, ro, se = bufs[bi]
            pltpu.make_async_copy(yref.at[si], ro, se).wait()
            pltpu.sync_copy(ro, acc.at[di], add=True)

        issue(base0, 0)

        @pl.loop(0, CPT - 2, step=2)
        def _(ci):
            b = base0 + ci * CHUNK
            issue(b + CHUNK, 1)
            drain_scatter(0)
            issue(b + 2 * CHUNK, 0)
            drain_scatter(1)

        issue(base0 + (CPT - 1) * CHUNK, 1)
        drain_scatter(0)
        drain_scatter(1)
        plsc.subcore_barrier()
        pltpu.sync_copy(acc.at[pl.ds(r0, RPT)],
                        out_hbm.at[v, core, pl.ds(r0, RPT)])


def _row_spec(cols):
    return pl.BlockSpec((RBLK, cols), lambda i: (i, 0))


def _stk_spec():
    return pl.BlockSpec((2, RBLK, H), lambda i: (0, i, 0))


def _w_spec():
    return pl.BlockSpec((D, D), lambda i: (0, 0))


def _b_spec():
    return pl.BlockSpec((1, D), lambda i: (0, 0))


def _tc1_body(x_ref, deg0_ref, deg1_ref, bW1_ref, bb1_ref, bW2_ref, bb2_ref,
              g0W_ref, g1W_ref, xb_ref, y0_ref, y1_ref, dinv0_ref, dinv1_ref):
    x = x_ref[...]
    dinv0 = lax.rsqrt(deg0_ref[...] + 1.0)
    dinv1 = lax.rsqrt(deg1_ref[...] + 1.0)
    dinv0_ref[...] = dinv0
    dinv1_ref[...] = dinv1
    h = jnp.maximum(jnp.dot(x, bW1_ref[...]) + bb1_ref[...], 0.0)
    xb_ref[...] = jnp.maximum(jnp.dot(h, bW2_ref[...]) + bb2_ref[...], 0.0)
    y0 = dinv0 * jnp.dot(x, g0W_ref[...])
    y1 = dinv1 * jnp.dot(x, g1W_ref[...])
    y0_ref[0] = y0[:, :H]
    y0_ref[1] = y0[:, H:]
    y1_ref[0] = y1[:, :H]
    y1_ref[1] = y1[:, H:]


def _tc2_body(A0a_ref, A0b_ref, A1a_ref, A1b_ref, y0_ref, y1_ref,
              dinv0_ref, dinv1_ref, g0b0_ref, g1b0_ref, g0W1_ref, g1W1_ref,
              y0n_ref, y1n_ref):
    for (Aa, Ab, y, dv, b, W, yn) in (
            (A0a_ref, A0b_ref, y0_ref, dinv0_ref, g0b0_ref, g0W1_ref, y0n_ref),
            (A1a_ref, A1b_ref, y1_ref, dinv1_ref, g1b0_ref, g1W1_ref, y1n_ref)):
        d = dv[...]
        A = jnp.concatenate([Aa[...] + y[0], Ab[...] + y[1]], axis=1)
        hcur = jnp.maximum(d * A + b[...], 0.0)
        ynew = d * jnp.dot(hcur, W[...])
        yn[0] = ynew[:, :H]
        yn[1] = ynew[:, H:]


def _tc3_body(A0a_ref, A0b_ref, A1a_ref, A1b_ref, y0_ref, y1_ref,
              dinv0_ref, dinv1_ref, g0b1_ref, g1b1_ref,
              d0W1_ref, d0b1_ref, d0W2_ref, d0b2_ref,
              d1W1_ref, d1b1_ref, d1W2_ref, d1b2_ref, xb_ref, out_ref):
    acc = xb_ref[...]
    for (Aa, Ab, y, dv, gb, W1, b1, W2, b2) in (
            (A0a_ref, A0b_ref, y0_ref, dinv0_ref, g0b1_ref,
             d0W1_ref, d0b1_ref, d0W2_ref, d0b2_ref),
            (A1a_ref, A1b_ref, y1_ref, dinv1_ref, g1b1_ref,
             d1W1_ref, d1b1_ref, d1W2_ref, d1b2_ref)):
        d = dv[...]
        A = jnp.concatenate([Aa[...] + y[0], Ab[...] + y[1]], axis=1)
        Hcur = jnp.maximum(d * A + gb[...], 0.0)
        h1 = jnp.maximum(jnp.dot(Hcur, W1[...]) + b1[...], 0.0)
        acc = acc + jnp.maximum(jnp.dot(h1, W2[...]) + b2[...], 0.0)
    out_ref[...] = acc


def kernel(x_initial, edge_index_0, edge_index_1,
           basic_W1, basic_b1, basic_W2, basic_b2,
           gnn0_W0, gnn0_b0, gnn0_W1, gnn0_b1,
           gnn1_W0, gnn1_b0, gnn1_W1, gnn1_b1,
           dec0_W1, dec0_b1, dec0_W2, dec0_b2,
           dec1_W1, dec1_b1, dec1_W2, dec1_b2):
    f32 = jnp.float32
    pad = E_PAD - E
    pad_src = (jnp.arange(pad, dtype=jnp.int32) * 997) % N
    pad_dst = N + (jnp.arange(pad, dtype=jnp.int32) % 240)

    def prep(ei):
        s = jnp.concatenate([ei[0].astype(jnp.int32), pad_src])
        dd = jnp.concatenate([ei[1].astype(jnp.int32), pad_dst])
        return jnp.stack([s, s + N]), dd

    srcs0, dst0 = prep(edge_index_0)
    srcs1, dst1 = prep(edge_index_1)

    zerosH = jnp.zeros((NROW, H), f32)
    ones1 = jnp.ones((DCH,), f32)
    zeros1 = jnp.zeros((NROW,), f32)

    deg_out = _deg_kernel(dst0, dst1, ones1, zeros1)
    deg0 = (deg_out[0, 0, :N] + deg_out[0, 1, :N])[:, None]
    deg1 = (deg_out[1, 0, :N] + deg_out[1, 1, :N])[:, None]

    b = lambda v: v.reshape(1, D)
    row_f = jax.ShapeDtypeStruct((N, D), f32)
    stk_f = jax.ShapeDtypeStruct((2, N, H), f32)
    d1_f = jax.ShapeDtypeStruct((N, 1), f32)

    xb, y0, y1, dinv0, dinv1 = pl.pallas_call(
        _tc1_body,
        grid=(GRID,),
        in_specs=[_row_spec(D), _row_spec(1), _row_spec(1),
                  _w_spec(), _b_spec(), _w_spec(), _b_spec(),
                  _w_spec(), _w_spec()],
        out_specs=[_row_spec(D), _stk_spec(), _stk_spec(),
                   _row_spec(1), _row_spec(1)],
        out_shape=[row_f, stk_f, stk_f, d1_f, d1_f],
    )(x_initial, deg0, deg1, basic_W1, b(basic_b1), basic_W2, b(basic_b2),
      gnn0_W0, gnn1_W0)

    def adj(y0s, y1s):
        out = _adj_kernel(srcs0, dst0, srcs1, dst1,
                          y0s.reshape(2 * N, H), y1s.reshape(2 * N, H), zerosH)
        return (out[0, 0, :N, :], out[0, 1, :N, :],
                out[1, 0, :N, :], out[1, 1, :N, :])

    A0a, A0b, A1a, A1b = adj(y0, y1)

    y0n, y1n = pl.pallas_call(
        _tc2_body,
        grid=(GRID,),
        in_specs=[_row_spec(H)] * 4 + [_stk_spec(), _stk_spec(),
                  _row_spec(1), _row_spec(1),
                  _b_spec(), _b_spec(), _w_spec(), _w_spec()],
        out_specs=[_stk_spec(), _stk_spec()],
        out_shape=[stk_f, stk_f],
    )(A0a, A0b, A1a, A1b, y0, y1, dinv0, dinv1,
      b(gnn0_b0), b(gnn1_b0), gnn0_W1, gnn1_W1)

    B0a, B0b, B1a, B1b = adj(y0n, y1n)

    x_total = pl.pallas_call(
        _tc3_body,
        grid=(GRID,),
        in_specs=[_row_spec(H)] * 4 + [_stk_spec(), _stk_spec(),
                  _row_spec(1), _row_spec(1), _b_spec(), _b_spec(),
                  _w_spec(), _b_spec(), _w_spec(), _b_spec(),
                  _w_spec(), _b_spec(), _w_spec(), _b_spec(),
                  _row_spec(D)],
        out_specs=_row_spec(D),
        out_shape=row_f,
    )(B0a, B0b, B1a, B1b, y0n, y1n, dinv0, dinv1,
      b(gnn0_b1), b(gnn1_b1),
      dec0_W1, b(dec0_b1), dec0_W2, b(dec0_b2),
      dec1_W1, b(dec1_b1), dec1_W2, b(dec1_b2), xb)

    return (x_total, jnp.float32(0.0))


# speedup vs baseline: 13.0255x; 1.0005x over previous
import functools

import jax
import jax.numpy as jnp
from jax import lax
from jax.experimental import pallas as pl
from jax.experimental.pallas import tpu as pltpu
from jax.experimental.pallas import tpu_sc as plsc

N = 10000
D = 256
H = 128
E = 160000
CHUNK = 128
NTILES = 16
NROW = N + 240
RPT = NROW // NTILES
E_PAD = 163840
NCH = E_PAD // CHUNK
CPT = NCH // NTILES
G = CPT // 2
RBLK = 1000
GRID = N // RBLK
DCH = 1024
DPT = (E_PAD // 2) // NTILES // DCH

_mesh = plsc.VectorSubcoreMesh(core_axis_name="c", subcore_axis_name="s")


@functools.partial(
    pl.kernel,
    out_type=jax.ShapeDtypeStruct((2, 2, NROW), jnp.float32),
    mesh=_mesh,
    scratch_types=[
        pltpu.VMEM((DCH,), jnp.int32),
        pltpu.VMEM((DCH,), jnp.float32),
        pltpu.VMEM_SHARED((NROW,), jnp.float32),
    ],
)
def _deg_kernel(dst0_hbm, dst1_hbm, ones_hbm, zeros_hbm, out_hbm,
                idx_v, ones_v, acc_sh):
    core = lax.axis_index("c")
    sub = lax.axis_index("s")
    r0 = sub * RPT
    pltpu.sync_copy(ones_hbm, ones_v)
    for v, dref in ((0, dst0_hbm), (1, dst1_hbm)):
        pltpu.sync_copy(zeros_hbm.at[pl.ds(r0, RPT)], acc_sh.at[pl.ds(r0, RPT)])
        plsc.subcore_barrier()
        base0 = (core * (E_PAD // 2)) + sub * (DPT * DCH)

        @pl.loop(0, DPT)
        def _(ci):
            b = base0 + ci * DCH
            pltpu.sync_copy(dref.at[pl.ds(b, DCH)], idx_v)
            pltpu.sync_copy(ones_v, acc_sh.at[idx_v], add=True)
        plsc.subcore_barrier()
        pltpu.sync_copy(acc_sh.at[pl.ds(r0, RPT)],
                        out_hbm.at[v, core, pl.ds(r0, RPT)])


@functools.partial(
    pl.kernel,
    out_type=jax.ShapeDtypeStruct((2, 2, NROW, H), jnp.float32),
    mesh=_mesh,
    scratch_types=[
        pltpu.VMEM((CHUNK,), jnp.int32),
        pltpu.VMEM((CHUNK,), jnp.int32),
        pltpu.VMEM((CHUNK,), jnp.int32),
        pltpu.VMEM((CHUNK,), jnp.int32),
        pltpu.VMEM((CHUNK, H), jnp.float32),
        pltpu.VMEM((CHUNK, H), jnp.float32),
        pltpu.VMEM_SHARED((NROW, H), jnp.float32),
        pltpu.SemaphoreType.DMA,
        pltpu.SemaphoreType.DMA,
    ],
)
def _adj_kernel(srcs0_hbm, dst0_hbm, srcs1_hbm, dst1_hbm, y0_hbm, y1_hbm,
                zeros_hbm, out_hbm, sidx0, didx0, sidx1, didx1,
                rows0, rows1, acc, sem0, sem1):
    core = lax.axis_index("c")
    sub = lax.axis_index("s")
    r0 = sub * RPT
    bufs = ((sidx0, didx0, rows0, sem0), (sidx1, didx1, rows1, sem1))
    for v, (sref, dref, yref) in enumerate(
            ((srcs0_hbm, dst0_hbm, y0_hbm), (srcs1_hbm, dst1_hbm, y1_hbm))):
        pltpu.sync_copy(zeros_hbm.at[pl.ds(r0, RPT)], acc.at[pl.ds(r0, RPT)])
        plsc.subcore_barrier()
        base0 = sub * (CPT * CHUNK)

        def issue(b, bi):
            si, di, ro, se = bufs[bi]
            pltpu.sync_copy(sref.at[core, pl.ds(b, CHUNK)], si)
            pltpu.sync_copy(dref.at[pl.ds(b, CHUNK)], di)
            pltpu.async_copy(yref.at[si], ro, se)

        def drain_scatter(bi):
            si, di, ro, se = bufs[bi]
            pltpu.make_async_copy(yref.at[si], ro, se).wait()
            pltpu.sync_copy(ro, acc.at[di], add=True)

        issue(base0, 0)

        @pl.loop(0, CPT - 2, step=2)
        def _(ci):
            b = base0 + ci * CHUNK
            issue(b + CHUNK, 1)
            drain_scatter(0)
            issue(b + 2 * CHUNK, 0)
            drain_scatter(1)

        issue(base0 + (CPT - 1) * CHUNK, 1)
        drain_scatter(0)
        drain_scatter(1)
        plsc.subcore_barrier()
        pltpu.sync_copy(acc.at[pl.ds(r0, RPT)],
                        out_hbm.at[v, core, pl.ds(r0, RPT)])


def _dot(a, w):
    return jnp.dot(a.astype(jnp.bfloat16), w.astype(jnp.bfloat16),
                   preferred_element_type=jnp.float32)


def _row_spec(cols):
    return pl.BlockSpec((RBLK, cols), lambda i: (i, 0))


def _stk_spec():
    return pl.BlockSpec((2, RBLK, H), lambda i: (0, i, 0))


def _w_spec():
    return pl.BlockSpec((D, D), lambda i: (0, 0))


def _b_spec():
    return pl.BlockSpec((1, D), lambda i: (0, 0))


def _tc1_body(x_ref, deg0_ref, deg1_ref, bW1_ref, bb1_ref, bW2_ref, bb2_ref,
              g0W_ref, g1W_ref, xb_ref, y0_ref, y1_ref, dinv0_ref, dinv1_ref):
    x = x_ref[...]
    dinv0 = lax.rsqrt(deg0_ref[...] + 1.0)
    dinv1 = lax.rsqrt(deg1_ref[...] + 1.0)
    dinv0_ref[...] = dinv0
    dinv1_ref[...] = dinv1
    h = jnp.maximum(_dot(x, bW1_ref[...]) + bb1_ref[...], 0.0)
    xb_ref[...] = jnp.maximum(_dot(h, bW2_ref[...]) + bb2_ref[...], 0.0)
    y0 = dinv0 * _dot(x, g0W_ref[...])
    y1 = dinv1 * _dot(x, g1W_ref[...])
    y0_ref[0] = y0[:, :H]
    y0_ref[1] = y0[:, H:]
    y1_ref[0] = y1[:, :H]
    y1_ref[1] = y1[:, H:]


def _tc2_body(A0a_ref, A0b_ref, A1a_ref, A1b_ref, y0_ref, y1_ref,
              dinv0_ref, dinv1_ref, g0b0_ref, g1b0_ref, g0W1_ref, g1W1_ref,
              y0n_ref, y1n_ref):
    for (Aa, Ab, y, dv, b, W, yn) in (
            (A0a_ref, A0b_ref, y0_ref, dinv0_ref, g0b0_ref, g0W1_ref, y0n_ref),
            (A1a_ref, A1b_ref, y1_ref, dinv1_ref, g1b0_ref, g1W1_ref, y1n_ref)):
        d = dv[...]
        A = jnp.concatenate([Aa[...] + y[0], Ab[...] + y[1]], axis=1)
        hcur = jnp.maximum(d * A + b[...], 0.0)
        ynew = d * _dot(hcur, W[...])
        yn[0] = ynew[:, :H]
        yn[1] = ynew[:, H:]


def _tc3_body(A0a_ref, A0b_ref, A1a_ref, A1b_ref, y0_ref, y1_ref,
              dinv0_ref, dinv1_ref, g0b1_ref, g1b1_ref,
              d0W1_ref, d0b1_ref, d0W2_ref, d0b2_ref,
              d1W1_ref, d1b1_ref, d1W2_ref, d1b2_ref, xb_ref, out_ref):
    acc = xb_ref[...]
    for (Aa, Ab, y, dv, gb, W1, b1, W2, b2) in (
            (A0a_ref, A0b_ref, y0_ref, dinv0_ref, g0b1_ref,
             d0W1_ref, d0b1_ref, d0W2_ref, d0b2_ref),
            (A1a_ref, A1b_ref, y1_ref, dinv1_ref, g1b1_ref,
             d1W1_ref, d1b1_ref, d1W2_ref, d1b2_ref)):
        d = dv[...]
        A = jnp.concatenate([Aa[...] + y[0], Ab[...] + y[1]], axis=1)
        Hcur = jnp.maximum(d * A + gb[...], 0.0)
        h1 = jnp.maximum(_dot(Hcur, W1[...]) + b1[...], 0.0)
        acc = acc + jnp.maximum(_dot(h1, W2[...]) + b2[...], 0.0)
    out_ref[...] = acc


def kernel(x_initial, edge_index_0, edge_index_1,
           basic_W1, basic_b1, basic_W2, basic_b2,
           gnn0_W0, gnn0_b0, gnn0_W1, gnn0_b1,
           gnn1_W0, gnn1_b0, gnn1_W1, gnn1_b1,
           dec0_W1, dec0_b1, dec0_W2, dec0_b2,
           dec1_W1, dec1_b1, dec1_W2, dec1_b2):
    f32 = jnp.float32
    pad = E_PAD - E
    pad_src = (jnp.arange(pad, dtype=jnp.int32) * 997) % N
    pad_dst = N + (jnp.arange(pad, dtype=jnp.int32) % 240)

    def prep(ei):
        s = jnp.concatenate([ei[0].astype(jnp.int32), pad_src])
        dd = jnp.concatenate([ei[1].astype(jnp.int32), pad_dst])
        return jnp.stack([s, s + N]), dd

    srcs0, dst0 = prep(edge_index_0)
    srcs1, dst1 = prep(edge_index_1)

    zerosH = jnp.zeros((NROW, H), f32)
    ones1 = jnp.ones((DCH,), f32)
    zeros1 = jnp.zeros((NROW,), f32)

    deg_out = _deg_kernel(dst0, dst1, ones1, zeros1)
    deg0 = (deg_out[0, 0, :N] + deg_out[0, 1, :N])[:, None]
    deg1 = (deg_out[1, 0, :N] + deg_out[1, 1, :N])[:, None]

    b = lambda v: v.reshape(1, D)
    row_f = jax.ShapeDtypeStruct((N, D), f32)
    stk_f = jax.ShapeDtypeStruct((2, N, H), f32)
    d1_f = jax.ShapeDtypeStruct((N, 1), f32)

    xb, y0, y1, dinv0, dinv1 = pl.pallas_call(
        _tc1_body,
        grid=(GRID,),
        in_specs=[_row_spec(D), _row_spec(1), _row_spec(1),
                  _w_spec(), _b_spec(), _w_spec(), _b_spec(),
                  _w_spec(), _w_spec()],
        out_specs=[_row_spec(D), _stk_spec(), _stk_spec(),
                   _row_spec(1), _row_spec(1)],
        out_shape=[row_f, stk_f, stk_f, d1_f, d1_f],
    )(x_initial, deg0, deg1, basic_W1, b(basic_b1), basic_W2, b(basic_b2),
      gnn0_W0, gnn1_W0)

    def adj(y0s, y1s):
        out = _adj_kernel(srcs0, dst0, srcs1, dst1,
                          y0s.reshape(2 * N, H), y1s.reshape(2 * N, H), zerosH)
        return (out[0, 0, :N, :], out[0, 1, :N, :],
                out[1, 0, :N, :], out[1, 1, :N, :])

    A0a, A0b, A1a, A1b = adj(y0, y1)

    y0n, y1n = pl.pallas_call(
        _tc2_body,
        grid=(GRID,),
        in_specs=[_row_spec(H)] * 4 + [_stk_spec(), _stk_spec(),
                  _row_spec(1), _row_spec(1),
                  _b_spec(), _b_spec(), _w_spec(), _w_spec()],
        out_specs=[_stk_spec(), _stk_spec()],
        out_shape=[stk_f, stk_f],
    )(A0a, A0b, A1a, A1b, y0, y1, dinv0, dinv1,
      b(gnn0_b0), b(gnn1_b0), gnn0_W1, gnn1_W1)

    B0a, B0b, B1a, B1b = adj(y0n, y1n)

    x_total = pl.pallas_call(
        _tc3_body,
        grid=(GRID,),
        in_specs=[_row_spec(H)] * 4 + [_stk_spec(), _stk_spec(),
                  _row_spec(1), _row_spec(1), _b_spec(), _b_spec(),
                  _w_spec(), _b_spec(), _w_spec(), _b_spec(),
                  _w_spec(), _b_spec(), _w_spec(), _b_spec(),
                  _row_spec(D)],
        out_specs=_row_spec(D),
        out_shape=row_f,
    )(B0a, B0b, B1a, B1b, y0n, y1n, dinv0, dinv1,
      b(gnn0_b1), b(gnn1_b1),
      dec0_W1, b(dec0_b1), dec0_W2, b(dec0_b2),
      dec1_W1, b(dec1_b1), dec1_W2, b(dec1_b2), xb)

    return (x_total, jnp.float32(0.0))


# speedup vs baseline: 13.4846x; 1.0352x over previous
import functools

import jax
import jax.numpy as jnp
from jax import lax
from jax.experimental import pallas as pl
from jax.experimental.pallas import tpu as pltpu
from jax.experimental.pallas import tpu_sc as plsc

N = 10000
D = 256
H = 128
E = 160000
CHUNK = 128
NTILES = 16
NROW = N + 240
RPT = NROW // NTILES
E_PAD = 163840
NCH = E_PAD // CHUNK
CPT = NCH // NTILES
G = CPT // 2
RBLK = 1000
GRID = N // RBLK
DCH = 1024
DPT = (E_PAD // 2) // NTILES // DCH

_mesh = plsc.VectorSubcoreMesh(core_axis_name="c", subcore_axis_name="s")


@functools.partial(
    pl.kernel,
    out_type=jax.ShapeDtypeStruct((2, 2, NROW), jnp.float32),
    mesh=_mesh,
    scratch_types=[
        pltpu.VMEM((DCH,), jnp.int32),
        pltpu.VMEM((DCH,), jnp.float32),
        pltpu.VMEM_SHARED((NROW,), jnp.float32),
    ],
)
def _deg_kernel(dst0_hbm, dst1_hbm, ones_hbm, zeros_hbm, out_hbm,
                idx_v, ones_v, acc_sh):
    core = lax.axis_index("c")
    sub = lax.axis_index("s")
    r0 = sub * RPT
    pltpu.sync_copy(ones_hbm, ones_v)
    for v, dref in ((0, dst0_hbm), (1, dst1_hbm)):
        pltpu.sync_copy(zeros_hbm.at[pl.ds(r0, RPT)], acc_sh.at[pl.ds(r0, RPT)])
        plsc.subcore_barrier()
        base0 = (core * (E_PAD // 2)) + sub * (DPT * DCH)

        @pl.loop(0, DPT)
        def _(ci):
            b = base0 + ci * DCH
            pltpu.sync_copy(dref.at[pl.ds(b, DCH)], idx_v)
            pltpu.sync_copy(ones_v, acc_sh.at[idx_v], add=True)
        plsc.subcore_barrier()
        pltpu.sync_copy(acc_sh.at[pl.ds(r0, RPT)],
                        out_hbm.at[v, core, pl.ds(r0, RPT)])


@functools.partial(
    pl.kernel,
    out_type=jax.ShapeDtypeStruct((2, 2, NROW, H), jnp.float32),
    mesh=_mesh,
    scratch_types=[
        pltpu.VMEM((CHUNK,), jnp.int32),
        pltpu.VMEM((CHUNK,), jnp.int32),
        pltpu.VMEM((CHUNK,), jnp.int32),
        pltpu.VMEM((CHUNK,), jnp.int32),
        pltpu.VMEM((CHUNK, H), jnp.float32),
        pltpu.VMEM((CHUNK, H), jnp.float32),
        pltpu.VMEM_SHARED((NROW, H), jnp.float32),
        pltpu.SemaphoreType.DMA,
        pltpu.SemaphoreType.DMA,
    ],
)
def _adj_kernel(srcs0_hbm, dst0_hbm, srcs1_hbm, dst1_hbm, y0_hbm, y1_hbm,
                zeros_hbm, out_hbm, sidx0, didx0, sidx1, didx1,
                rows0, rows1, acc, sem0, sem1):
    core = lax.axis_index("c")
    sub = lax.axis_index("s")
    r0 = sub * RPT
    bufs = ((sidx0, didx0, rows0, sem0), (sidx1, didx1, rows1, sem1))
    for v, (sref, dref, yref) in enumerate(
            ((srcs0_hbm, dst0_hbm, y0_hbm), (srcs1_hbm, dst1_hbm, y1_hbm))):
        pltpu.sync_copy(zeros_hbm.at[pl.ds(r0, RPT)], acc.at[pl.ds(r0, RPT)])
        plsc.subcore_barrier()
        base0 = sub * (CPT * CHUNK)

        def issue(b, bi):
            si, di, ro, se = bufs[bi]
            pltpu.sync_copy(sref.at[core, pl.ds(b, CHUNK)], si)
            pltpu.sync_copy(dref.at[pl.ds(b, CHUNK)], di)
            pltpu.async_copy(yref.at[si], ro, se)

        def drain_scatter(bi):
            si, di, ro, se = bufs[bi]
            pltpu.make_async_copy(yref.at[si], ro, se).wait()
            pltpu.sync_copy(ro, acc.at[di], add=True)

        issue(base0, 0)

        @pl.loop(0, CPT - 2, step=2)
        def _(ci):
            b = base0 + ci * CHUNK
            issue(b + CHUNK, 1)
            drain_scatter(0)
            issue(b + 2 * CHUNK, 0)
            drain_scatter(1)

        issue(base0 + (CPT - 1) * CHUNK, 1)
        drain_scatter(0)
        drain_scatter(1)
        plsc.subcore_barrier()
        pltpu.sync_copy(acc.at[pl.ds(r0, RPT)],
                        out_hbm.at[v, core, pl.ds(r0, RPT)])


def _dot(a, w):
    return jnp.dot(a.astype(jnp.bfloat16), w.astype(jnp.bfloat16),
                   preferred_element_type=jnp.float32)


def _row_spec(cols):
    return pl.BlockSpec((RBLK, cols), lambda i: (i, 0))


def _a_spec(v, c):
    return pl.BlockSpec((1, 1, RBLK, H), lambda i, v=v, c=c: (v, c, i, 0))


def _stk_spec():
    return pl.BlockSpec((2, RBLK, H), lambda i: (0, i, 0))


def _w_spec():
    return pl.BlockSpec((D, D), lambda i: (0, 0))


def _b_spec():
    return pl.BlockSpec((1, D), lambda i: (0, 0))


def _tc1_body(x_ref, deg0_ref, deg1_ref, bW1_ref, bb1_ref, bW2_ref, bb2_ref,
              g0W_ref, g1W_ref, xb_ref, y0_ref, y1_ref, dinv0_ref, dinv1_ref):
    x = x_ref[...]
    dinv0 = lax.rsqrt(deg0_ref[...] + 1.0)
    dinv1 = lax.rsqrt(deg1_ref[...] + 1.0)
    dinv0_ref[...] = dinv0
    dinv1_ref[...] = dinv1
    h = jnp.maximum(_dot(x, bW1_ref[...]) + bb1_ref[...], 0.0)
    xb_ref[...] = jnp.maximum(_dot(h, bW2_ref[...]) + bb2_ref[...], 0.0)
    y0 = dinv0 * _dot(x, g0W_ref[...])
    y1 = dinv1 * _dot(x, g1W_ref[...])
    y0_ref[0] = y0[:, :H]
    y0_ref[1] = y0[:, H:]
    y1_ref[0] = y1[:, :H]
    y1_ref[1] = y1[:, H:]


def _tc2_body(A0a_ref, A0b_ref, A1a_ref, A1b_ref, y0_ref, y1_ref,
              dinv0_ref, dinv1_ref, g0b0_ref, g1b0_ref, g0W1_ref, g1W1_ref,
              y0n_ref, y1n_ref):
    for (Aa, Ab, y, dv, b, W, yn) in (
            (A0a_ref, A0b_ref, y0_ref, dinv0_ref, g0b0_ref, g0W1_ref, y0n_ref),
            (A1a_ref, A1b_ref, y1_ref, dinv1_ref, g1b0_ref, g1W1_ref, y1n_ref)):
        d = dv[...]
        A = jnp.concatenate([Aa[0, 0] + y[0], Ab[0, 0] + y[1]], axis=1)
        hcur = jnp.maximum(d * A + b[...], 0.0)
        ynew = d * _dot(hcur, W[...])
        yn[0] = ynew[:, :H]
        yn[1] = ynew[:, H:]


def _tc3_body(A0a_ref, A0b_ref, A1a_ref, A1b_ref, y0_ref, y1_ref,
              dinv0_ref, dinv1_ref, g0b1_ref, g1b1_ref,
              d0W1_ref, d0b1_ref, d0W2_ref, d0b2_ref,
              d1W1_ref, d1b1_ref, d1W2_ref, d1b2_ref, xb_ref, out_ref):
    acc = xb_ref[...]
    for (Aa, Ab, y, dv, gb, W1, b1, W2, b2) in (
            (A0a_ref, A0b_ref, y0_ref, dinv0_ref, g0b1_ref,
             d0W1_ref, d0b1_ref, d0W2_ref, d0b2_ref),
            (A1a_ref, A1b_ref, y1_ref, dinv1_ref, g1b1_ref,
             d1W1_ref, d1b1_ref, d1W2_ref, d1b2_ref)):
        d = dv[...]
        A = jnp.concatenate([Aa[0, 0] + y[0], Ab[0, 0] + y[1]], axis=1)
        Hcur = jnp.maximum(d * A + gb[...], 0.0)
        h1 = jnp.maximum(_dot(Hcur, W1[...]) + b1[...], 0.0)
        acc = acc + jnp.maximum(_dot(h1, W2[...]) + b2[...], 0.0)
    out_ref[...] = acc


def kernel(x_initial, edge_index_0, edge_index_1,
           basic_W1, basic_b1, basic_W2, basic_b2,
           gnn0_W0, gnn0_b0, gnn0_W1, gnn0_b1,
           gnn1_W0, gnn1_b0, gnn1_W1, gnn1_b1,
           dec0_W1, dec0_b1, dec0_W2, dec0_b2,
           dec1_W1, dec1_b1, dec1_W2, dec1_b2):
    f32 = jnp.float32
    pad = E_PAD - E
    pad_src = (jnp.arange(pad, dtype=jnp.int32) * 997) % N
    pad_dst = N + (jnp.arange(pad, dtype=jnp.int32) % 240)

    def prep(ei):
        s = jnp.concatenate([ei[0].astype(jnp.int32), pad_src])
        dd = jnp.concatenate([ei[1].astype(jnp.int32), pad_dst])
        return jnp.stack([s, s + N]), dd

    srcs0, dst0 = prep(edge_index_0)
    srcs1, dst1 = prep(edge_index_1)

    zerosH = jnp.zeros((NROW, H), f32)
    ones1 = jnp.ones((DCH,), f32)
    zeros1 = jnp.zeros((NROW,), f32)

    deg_out = _deg_kernel(dst0, dst1, ones1, zeros1)
    deg0 = (deg_out[0, 0, :N] + deg_out[0, 1, :N])[:, None]
    deg1 = (deg_out[1, 0, :N] + deg_out[1, 1, :N])[:, None]

    b = lambda v: v.reshape(1, D)
    row_f = jax.ShapeDtypeStruct((N, D), f32)
    stk_f = jax.ShapeDtypeStruct((2, N, H), f32)
    d1_f = jax.ShapeDtypeStruct((N, 1), f32)

    xb, y0, y1, dinv0, dinv1 = pl.pallas_call(
        _tc1_body,
        grid=(GRID,),
        in_specs=[_row_spec(D), _row_spec(1), _row_spec(1),
                  _w_spec(), _b_spec(), _w_spec(), _b_spec(),
                  _w_spec(), _w_spec()],
        out_specs=[_row_spec(D), _stk_spec(), _stk_spec(),
                   _row_spec(1), _row_spec(1)],
        out_shape=[row_f, stk_f, stk_f, d1_f, d1_f],
    )(x_initial, deg0, deg1, basic_W1, b(basic_b1), basic_W2, b(basic_b2),
      gnn0_W0, gnn1_W0)

    def adj(y0s, y1s):
        return _adj_kernel(srcs0, dst0, srcs1, dst1,
                           y0s.reshape(2 * N, H), y1s.reshape(2 * N, H),
                           zerosH)

    a_specs = [_a_spec(0, 0), _a_spec(0, 1), _a_spec(1, 0), _a_spec(1, 1)]
    A = adj(y0, y1)

    y0n, y1n = pl.pallas_call(
        _tc2_body,
        grid=(GRID,),
        in_specs=a_specs + [_stk_spec(), _stk_spec(),
                  _row_spec(1), _row_spec(1),
                  _b_spec(), _b_spec(), _w_spec(), _w_spec()],
        out_specs=[_stk_spec(), _stk_spec()],
        out_shape=[stk_f, stk_f],
    )(A, A, A, A, y0, y1, dinv0, dinv1,
      b(gnn0_b0), b(gnn1_b0), gnn0_W1, gnn1_W1)

    B = adj(y0n, y1n)

    x_total = pl.pallas_call(
        _tc3_body,
        grid=(GRID,),
        in_specs=a_specs + [_stk_spec(), _stk_spec(),
                  _row_spec(1), _row_spec(1), _b_spec(), _b_spec(),
                  _w_spec(), _b_spec(), _w_spec(), _b_spec(),
                  _w_spec(), _b_spec(), _w_spec(), _b_spec(),
                  _row_spec(D)],
        out_specs=_row_spec(D),
        out_shape=row_f,
    )(B, B, B, B, y0n, y1n, dinv0, dinv1,
      b(gnn0_b1), b(gnn1_b1),
      dec0_W1, b(dec0_b1), dec0_W2, b(dec0_b2),
      dec1_W1, b(dec1_b1), dec1_W2, b(dec1_b2), xb)

    return (x_total, jnp.float32(0.0))


# speedup vs baseline: 13.5980x; 1.0084x over previous
import functools

import jax
import jax.numpy as jnp
from jax import lax
from jax.experimental import pallas as pl
from jax.experimental.pallas import tpu as pltpu
from jax.experimental.pallas import tpu_sc as plsc

N = 10000
D = 256
H = 128
E = 160000
CHUNK = 128
NTILES = 16
NROW = N + 240
RPT = NROW // NTILES
E_PAD = 163840
NCH = E_PAD // CHUNK
CPT = NCH // NTILES
G = CPT // 2
RBLK = 1000
GRID = N // RBLK
DCH = 1024
DPT = (E_PAD // 2) // NTILES // DCH

_mesh = plsc.VectorSubcoreMesh(core_axis_name="c", subcore_axis_name="s")


@functools.partial(
    pl.kernel,
    out_type=jax.ShapeDtypeStruct((2, 2, NROW), jnp.float32),
    mesh=_mesh,
    scratch_types=[
        pltpu.VMEM((DCH,), jnp.int32),
        pltpu.VMEM((DCH,), jnp.float32),
        pltpu.VMEM_SHARED((NROW,), jnp.float32),
    ],
)
def _deg_kernel(dst0_hbm, dst1_hbm, ones_hbm, zeros_hbm, out_hbm,
                idx_v, ones_v, acc_sh):
    core = lax.axis_index("c")
    sub = lax.axis_index("s")
    r0 = sub * RPT
    pltpu.sync_copy(ones_hbm, ones_v)
    for v, dref in ((0, dst0_hbm), (1, dst1_hbm)):
        pltpu.sync_copy(zeros_hbm.at[pl.ds(r0, RPT)], acc_sh.at[pl.ds(r0, RPT)])
        plsc.subcore_barrier()
        base0 = (core * (E_PAD // 2)) + sub * (DPT * DCH)

        @pl.loop(0, DPT)
        def _(ci):
            b = base0 + ci * DCH
            pltpu.sync_copy(dref.at[pl.ds(b, DCH)], idx_v)
            pltpu.sync_copy(ones_v, acc_sh.at[idx_v], add=True)
        plsc.subcore_barrier()
        pltpu.sync_copy(acc_sh.at[pl.ds(r0, RPT)],
                        out_hbm.at[v, core, pl.ds(r0, RPT)])


@functools.partial(
    pl.kernel,
    out_type=jax.ShapeDtypeStruct((2, NROW, H), jnp.float32),
    mesh=_mesh,
    scratch_types=[
        pltpu.VMEM((CHUNK,), jnp.int32),
        pltpu.VMEM((CHUNK,), jnp.int32),
        pltpu.VMEM((CHUNK,), jnp.int32),
        pltpu.VMEM((CHUNK,), jnp.int32),
        pltpu.VMEM((CHUNK, H), jnp.float32),
        pltpu.VMEM((CHUNK, H), jnp.float32),
        pltpu.VMEM_SHARED((NROW, H), jnp.float32),
        pltpu.SemaphoreType.DMA,
        pltpu.SemaphoreType.DMA,
    ],
)
def _adj_kernel(sref, dref, yref, zeros_hbm, out_hbm,
                sidx0, didx0, sidx1, didx1, rows0, rows1, acc, sem0, sem1):
    core = lax.axis_index("c")
    sub = lax.axis_index("s")
    r0 = sub * RPT
    bufs = ((sidx0, didx0, rows0, sem0), (sidx1, didx1, rows1, sem1))
    pltpu.sync_copy(zeros_hbm.at[pl.ds(r0, RPT)], acc.at[pl.ds(r0, RPT)])
    plsc.subcore_barrier()
    base0 = sub * (CPT * CHUNK)

    def issue(b, bi):
        si, di, ro, se = bufs[bi]
        pltpu.sync_copy(sref.at[core, pl.ds(b, CHUNK)], si)
        pltpu.sync_copy(dref.at[pl.ds(b, CHUNK)], di)
        pltpu.async_copy(yref.at[si], ro, se)

    def drain_scatter(bi):
        si, di, ro, se = bufs[bi]
        pltpu.make_async_copy(yref.at[si], ro, se).wait()
        pltpu.sync_copy(ro, acc.at[di], add=True)

    issue(base0, 0)

    @pl.loop(0, CPT - 2, step=2)
    def _(ci):
        b = base0 + ci * CHUNK
        issue(b + CHUNK, 1)
        drain_scatter(0)
        issue(b + 2 * CHUNK, 0)
        drain_scatter(1)

    issue(base0 + (CPT - 1) * CHUNK, 1)
    drain_scatter(0)
    drain_scatter(1)
    plsc.subcore_barrier()
    pltpu.sync_copy(acc.at[pl.ds(r0, RPT)],
                    out_hbm.at[core, pl.ds(r0, RPT)])


def _dot(a, w):
    return jnp.dot(a.astype(jnp.bfloat16), w.astype(jnp.bfloat16),
                   preferred_element_type=jnp.float32)


def _row_spec(cols):
    return pl.BlockSpec((RBLK, cols), lambda i: (i, 0))


def _av_spec(c):
    return pl.BlockSpec((1, RBLK, H), lambda i, c=c: (c, i, 0))


def _stk_spec():
    return pl.BlockSpec((2, RBLK, H), lambda i: (0, i, 0))


def _w_spec():
    return pl.BlockSpec((D, D), lambda i: (0, 0))


def _b_spec():
    return pl.BlockSpec((1, D), lambda i: (0, 0))


def _tc1_body(x_ref, deg0_ref, deg1_ref, bW1_ref, bb1_ref, bW2_ref, bb2_ref,
              g0W_ref, g1W_ref, xb_ref, y0_ref, y1_ref, dinv0_ref, dinv1_ref):
    x = x_ref[...]
    dinv0 = lax.rsqrt(deg0_ref[...] + 1.0)
    dinv1 = lax.rsqrt(deg1_ref[...] + 1.0)
    dinv0_ref[...] = dinv0
    dinv1_ref[...] = dinv1
    h = jnp.maximum(_dot(x, bW1_ref[...]) + bb1_ref[...], 0.0)
    xb_ref[...] = jnp.maximum(_dot(h, bW2_ref[...]) + bb2_ref[...], 0.0)
    y0 = dinv0 * _dot(x, g0W_ref[...])
    y1 = dinv1 * _dot(x, g1W_ref[...])
    y0_ref[0] = y0[:, :H]
    y0_ref[1] = y0[:, H:]
    y1_ref[0] = y1[:, :H]
    y1_ref[1] = y1[:, H:]


def _tc2v_body(Aa_ref, Ab_ref, y_ref, dv_ref, b_ref, W_ref, yn_ref):
    d = dv_ref[...]
    A = jnp.concatenate([Aa_ref[0] + y_ref[0], Ab_ref[0] + y_ref[1]], axis=1)
    hcur = jnp.maximum(d * A + b_ref[...], 0.0)
    ynew = d * _dot(hcur, W_ref[...])
    yn_ref[0] = ynew[:, :H]
    yn_ref[1] = ynew[:, H:]


def _tc3v_body(Aa_ref, Ab_ref, y_ref, dv_ref, gb_ref, W1_ref, b1_ref,
               W2_ref, b2_ref, acc_ref, out_ref):
    d = dv_ref[...]
    A = jnp.concatenate([Aa_ref[0] + y_ref[0], Ab_ref[0] + y_ref[1]], axis=1)
    Hcur = jnp.maximum(d * A + gb_ref[...], 0.0)
    h1 = jnp.maximum(_dot(Hcur, W1_ref[...]) + b1_ref[...], 0.0)
    out_ref[...] = acc_ref[...] + jnp.maximum(
        _dot(h1, W2_ref[...]) + b2_ref[...], 0.0)


def kernel(x_initial, edge_index_0, edge_index_1,
           basic_W1, basic_b1, basic_W2, basic_b2,
           gnn0_W0, gnn0_b0, gnn0_W1, gnn0_b1,
           gnn1_W0, gnn1_b0, gnn1_W1, gnn1_b1,
           dec0_W1, dec0_b1, dec0_W2, dec0_b2,
           dec1_W1, dec1_b1, dec1_W2, dec1_b2):
    f32 = jnp.float32
    pad = E_PAD - E
    pad_src = (jnp.arange(pad, dtype=jnp.int32) * 997) % N
    pad_dst = N + (jnp.arange(pad, dtype=jnp.int32) % 240)

    def prep(ei):
        s = jnp.concatenate([ei[0].astype(jnp.int32), pad_src])
        dd = jnp.concatenate([ei[1].astype(jnp.int32), pad_dst])
        return jnp.stack([s, s + N]), dd

    srcs0, dst0 = prep(edge_index_0)
    srcs1, dst1 = prep(edge_index_1)

    zerosH = jnp.zeros((NROW, H), f32)
    ones1 = jnp.ones((DCH,), f32)
    zeros1 = jnp.zeros((NROW,), f32)

    deg_out = _deg_kernel(dst0, dst1, ones1, zeros1)
    deg0 = (deg_out[0, 0, :N] + deg_out[0, 1, :N])[:, None]
    deg1 = (deg_out[1, 0, :N] + deg_out[1, 1, :N])[:, None]

    b = lambda v: v.reshape(1, D)
    row_f = jax.ShapeDtypeStruct((N, D), f32)
    stk_f = jax.ShapeDtypeStruct((2, N, H), f32)
    d1_f = jax.ShapeDtypeStruct((N, 1), f32)

    xb, y0, y1, dinv0, dinv1 = pl.pallas_call(
        _tc1_body,
        grid=(GRID,),
        in_specs=[_row_spec(D), _row_spec(1), _row_spec(1),
                  _w_spec(), _b_spec(), _w_spec(), _b_spec(),
                  _w_spec(), _w_spec()],
        out_specs=[_row_spec(D), _stk_spec(), _stk_spec(),
                   _row_spec(1), _row_spec(1)],
        out_shape=[row_f, stk_f, stk_f, d1_f, d1_f],
    )(x_initial, deg0, deg1, basic_W1, b(basic_b1), basic_W2, b(basic_b2),
      gnn0_W0, gnn1_W0)

    def adj(s, dd, ys):
        return _adj_kernel(s, dd, ys.reshape(2 * N, H), zerosH)

    def tc2v(A, y, dv, bb, W):
        return pl.pallas_call(
            _tc2v_body,
            grid=(GRID,),
            in_specs=[_av_spec(0), _av_spec(1), _stk_spec(), _row_spec(1),
                      _b_spec(), _w_spec()],
            out_specs=_stk_spec(),
            out_shape=stk_f,
        )(A, A, y, dv, b(bb), W)

    def tc3v(B, y, dv, gb, W1, b1, W2, b2, accin):
        return pl.pallas_call(
            _tc3v_body,
            grid=(GRID,),
            in_specs=[_av_spec(0), _av_spec(1), _stk_spec(), _row_spec(1),
                      _b_spec(), _w_spec(), _b_spec(), _w_spec(), _b_spec(),
                      _row_spec(D)],
            out_specs=_row_spec(D),
            out_shape=row_f,
        )(B, B, y, dv, b(gb), W1, b(b1), W2, b(b2), accin)

    A0 = adj(srcs0, dst0, y0)
    A1 = adj(srcs1, dst1, y1)
    y0n = tc2v(A0, y0, dinv0, gnn0_b0, gnn0_W1)
    y1n = tc2v(A1, y1, dinv1, gnn1_b0, gnn1_W1)
    B0 = adj(srcs0, dst0, y0n)
    B1 = adj(srcs1, dst1, y1n)
    t0 = tc3v(B0, y0n, dinv0, gnn0_b1, dec0_W1, dec0_b1, dec0_W2, dec0_b2, xb)
    x_total = tc3v(B1, y1n, dinv1, gnn1_b1,
                   dec1_W1, dec1_b1, dec1_W2, dec1_b2, t0)

    return (x_total, jnp.float32(0.0))
